# Initial kernel scaffold; baseline (speedup 1.0000x reference)
import jax
import jax.numpy as jnp
from jax.experimental import pallas as pl


def kernel(x, coeffs, bias, knots):
    raise NotImplementedError("write your pallas kernel here")


# speedup vs baseline: 51.6877x; 51.6877x over previous
import functools

import jax
import jax.numpy as jnp
from jax import lax
from jax.experimental import pallas as pl
from jax.experimental.pallas import tpu as pltpu
from jax.experimental.pallas import tpu_sc as plsc

B = 16384
D_IN = 64
D_OUT = 64
K = 32
XMIN = -3.0
XMAX = 3.0
HSTEP = (XMAX - XMIN) / (K - 1)
INV_H = (K - 1) / (XMAX - XMIN)

NW = 32
BPT = B // NW
IC = 16
NIC = D_IN // IC
BC = 128
NBC = BPT // BC
TW = K * D_OUT


def _slopes_body(c2_ref, knots_ref, slopes_ref):
    kn = knots_ref[...]
    h = kn[:, 1:] - kn[:, :-1]
    y = c2_ref[...]
    delta = (y[:, 1:] - y[:, :-1]) / (h + 1e-12)
    d_first = delta[:, :1]
    d_last = delta[:, -1:]
    dp = delta[:, :-1]
    dn = delta[:, 1:]
    same = dp * dn > 0
    w1v = 2.0 * h[:, 1:] + h[:, :-1]
    w2v = h[:, 1:] + 2.0 * h[:, :-1]
    d_int = (w1v + w2v) / (w1v / (dp + 1e-12) + w2v / (dn + 1e-12) + 1e-12)
    d_mid = jnp.where(same, d_int, jnp.zeros_like(d_int))
    slopes_ref[...] = jnp.concatenate([d_first, d_mid, d_last], axis=1)


def _weights_body(x_ref, j_ref, w0_ref, w1_ref, w2_ref, w3_ref):
    x = x_ref[...]
    xc = jnp.clip(x, XMIN, XMAX)
    u = (xc - XMIN) * INV_H
    jf = jnp.clip(jnp.floor(u), 0.0, float(K - 2))
    t = u - jf
    t2 = t * t
    t3 = t2 * t
    hh = HSTEP + 1e-12
    wy0 = 2.0 * t3 - 3.0 * t2 + 1.0
    wd0 = (t3 - 2.0 * t2 + t) * hh
    wy1 = -2.0 * t3 + 3.0 * t2
    wd1 = (t3 - t2) * hh
    below = x < XMIN
    above = x > XMAX
    zero = jnp.zeros_like(x)
    one = jnp.ones_like(x)
    wy0 = jnp.where(below, one, jnp.where(above, zero, wy0))
    wd0 = jnp.where(below, x - XMIN, jnp.where(above, zero, wd0))
    wy1 = jnp.where(below, zero, jnp.where(above, one, wy1))
    wd1 = jnp.where(below, zero, jnp.where(above, x - XMAX, wd1))
    jq = jnp.where(below, 0.0, jnp.where(above, float(K - 2), jf))
    j_ref[...] = jq.astype(jnp.int32)
    w0_ref[...] = wy0
    w1_ref[...] = wd0
    w2_ref[...] = wy1
    w3_ref[...] = wd1


_slopes_call = pl.pallas_call(
    _slopes_body,
    out_shape=jax.ShapeDtypeStruct((D_OUT * D_IN, K), jnp.float32),
)

BBLK = 2048
_weights_call = pl.pallas_call(
    _weights_body,
    grid=(B // BBLK,),
    in_specs=[pl.BlockSpec((BBLK, D_IN), lambda m: (m, 0))],
    out_specs=[pl.BlockSpec((BBLK, D_IN), lambda m: (m, 0))] * 5,
    out_shape=[
        jax.ShapeDtypeStruct((B, D_IN), jnp.int32),
        jax.ShapeDtypeStruct((B, D_IN), jnp.float32),
        jax.ShapeDtypeStruct((B, D_IN), jnp.float32),
        jax.ShapeDtypeStruct((B, D_IN), jnp.float32),
        jax.ShapeDtypeStruct((B, D_IN), jnp.float32),
    ],
)


def _sc_body(ytab_hbm, dtab_hbm, j_hbm, w0_hbm, w1_hbm, w2_hbm, w3_hbm,
             bias_hbm, out_hbm,
             ytab_v, dtab_v, j_v, w0_v, w1_v, w2_v, w3_v, bias_v, acc_v):
    wid = lax.axis_index("s") * 2 + lax.axis_index("c")
    b_base = wid * BPT
    pltpu.sync_copy(bias_hbm, bias_v)
    for ic in range(NIC):
        pltpu.sync_copy(ytab_hbm.at[pl.ds(ic * IC * TW, IC * TW)], ytab_v)
        pltpu.sync_copy(dtab_hbm.at[pl.ds(ic * IC * TW, IC * TW)], dtab_v)

        def bc_body(bc, _, ic=ic):
            b0 = b_base + bc * BC
            pltpu.sync_copy(j_hbm.at[pl.ds(b0, BC), pl.ds(ic * IC, IC)], j_v)
            pltpu.sync_copy(w0_hbm.at[pl.ds(b0, BC), pl.ds(ic * IC, IC)], w0_v)
            pltpu.sync_copy(w1_hbm.at[pl.ds(b0, BC), pl.ds(ic * IC, IC)], w1_v)
            pltpu.sync_copy(w2_hbm.at[pl.ds(b0, BC), pl.ds(ic * IC, IC)], w2_v)
            pltpu.sync_copy(w3_hbm.at[pl.ds(b0, BC), pl.ds(ic * IC, IC)], w3_v)

            def b_body(b, _, ic=ic, bc=bc):
                abase = (bc * BC + b) * D_OUT
                j_row = j_v[b, pl.ds(0, IC)]
                w0_row = w0_v[b, pl.ds(0, IC)]
                w1_row = w1_v[b, pl.ds(0, IC)]
                w2_row = w2_v[b, pl.ds(0, IC)]
                w3_row = w3_v[b, pl.ds(0, IC)]
                if ic == 0:
                    accs = [bias_v[pl.ds(c * 16, 16)] for c in range(4)]
                else:
                    accs = [acc_v[pl.ds(abase + c * 16, 16)]
                            for c in range(4)]
                for i in range(IC):
                    off = i * TW + j_row[i] * D_OUT
                    wy0 = w0_row[i]
                    wd0 = w1_row[i]
                    wy1 = w2_row[i]
                    wd1 = w3_row[i]
                    for c in range(4):
                        accs[c] = (accs[c]
                                   + wy0 * ytab_v[pl.ds(off + c * 16, 16)]
                                   + wd0 * dtab_v[pl.ds(off + c * 16, 16)]
                                   + wy1 * ytab_v[pl.ds(off + 64 + c * 16, 16)]
                                   + wd1 * dtab_v[pl.ds(off + 64 + c * 16, 16)])
                for c in range(4):
                    acc_v[pl.ds(abase + c * 16, 16)] = accs[c]
                return 0

            lax.fori_loop(0, BC, b_body, 0)
            return 0

        lax.fori_loop(0, NBC, bc_body, 0)
    pltpu.sync_copy(acc_v, out_hbm.at[pl.ds(b_base * D_OUT, BPT * D_OUT)])


_sc = pl.kernel(
    _sc_body,
    out_type=jax.ShapeDtypeStruct((B * D_OUT,), jnp.float32),
    mesh=plsc.VectorSubcoreMesh(core_axis_name="c", subcore_axis_name="s"),
    compiler_params=pltpu.CompilerParams(use_tc_tiling_on_sc=False),
    scratch_types=[
        pltpu.VMEM((IC * TW,), jnp.float32),
        pltpu.VMEM((IC * TW,), jnp.float32),
        pltpu.VMEM((BC, IC), jnp.int32),
        pltpu.VMEM((BC, IC), jnp.float32),
        pltpu.VMEM((BC, IC), jnp.float32),
        pltpu.VMEM((BC, IC), jnp.float32),
        pltpu.VMEM((BC, IC), jnp.float32),
        pltpu.VMEM((D_OUT,), jnp.float32),
        pltpu.VMEM((BPT * D_OUT,), jnp.float32),
    ],
)


def kernel(x, coeffs, bias, knots):
    c2 = coeffs.reshape(D_OUT * D_IN, K)
    knots2 = knots.reshape(1, K)
    slopes2 = _slopes_call(c2, knots2)
    jidx, w0, w1, w2, w3 = _weights_call(x)
    ytab = coeffs.transpose(1, 2, 0).reshape(D_IN * K * D_OUT)
    dtab = (slopes2.reshape(D_OUT, D_IN, K).transpose(1, 2, 0)
            .reshape(D_IN * K * D_OUT))
    out = _sc(ytab, dtab, jidx, w0, w1, w2, w3, bias)
    return out.reshape(B, D_OUT)
